# baseline (speedup 1.0000x reference)
import functools
import math

import jax
import jax.numpy as jnp
from jax.experimental import pallas as pl
from jax.experimental.pallas import tpu as pltpu

B = 1
L = 4096
QUERY_DIM = 1024
HEADS = 16
DIM_HEAD = 64
NUM_FRAMES = 8
NUM_SPATIAL = 12
NUM_TEMPORAL = 4
MAX_SPATIAL_LEN = 2048
EPS = 1e-6

T = NUM_FRAMES
S = L // T
INNER = HEADS * DIM_HEAD

HG = 4
G = HEADS // HG
HGD = HG * DIM_HEAD

ROW_BLK = 512


def _seq_shifts(n, max_len):
    shifts = [0]
    s = 1
    while len(shifts) < n and s < max_len:
        shifts.append(s)
        if len(shifts) < n:
            shifts.append(-s)
        s *= 2
    return shifts[:n]


def _temporal_shifts(n):
    shifts = []
    s = 1
    while len(shifts) < n:
        shifts.append(s)
        if len(shifts) < n:
            shifts.append(-s)
        s *= 2
    return shifts[:n]


SPATIAL_SHIFTS = _seq_shifts(NUM_SPATIAL, MAX_SPATIAL_LEN)
TEMPORAL_SHIFTS = _temporal_shifts(NUM_TEMPORAL)
TOTAL_EDGES = NUM_SPATIAL + NUM_TEMPORAL


def _proj_kernel(x_ref, wq_ref, wk_ref, wv_ref, bq_ref, bk_ref, bv_ref,
                 qnw_ref, knw_ref, q_ref, k_ref, v_ref):
    xb = x_ref[...]
    q = jnp.dot(xb, wq_ref[...], preferred_element_type=jnp.float32) + bq_ref[...]
    k = jnp.dot(xb, wk_ref[...], preferred_element_type=jnp.float32) + bk_ref[...]
    v = jnp.dot(xb, wv_ref[...], preferred_element_type=jnp.float32) + bv_ref[...]
    qm = jnp.mean(q * q, axis=-1, keepdims=True)
    km = jnp.mean(k * k, axis=-1, keepdims=True)
    q_ref[...] = q * jax.lax.rsqrt(qm + EPS) * qnw_ref[...]
    k_ref[...] = k * jax.lax.rsqrt(km + EPS) * knw_ref[...]
    v_ref[...] = v


def _shift_rows(x, s):
    n = x.shape[0]
    s = s % n
    if s == 0:
        return x
    hi = jax.lax.slice_in_dim(x, s, n, axis=0)
    lo = jax.lax.slice_in_dim(x, 0, s, axis=0)
    return jax.lax.concatenate([hi, lo], dimension=0)


EC = TOTAL_EDGES * HG


def _edge_masks():
    d_idx = jax.lax.broadcasted_iota(jnp.int32, (HGD, EC), 0)
    c_idx = jax.lax.broadcasted_iota(jnp.int32, (HGD, EC), 1)
    head_match = (d_idx // DIM_HEAD) == (c_idx % HG)
    masks = []
    for e in range(TOTAL_EDGES):
        m = (head_match & (c_idx // HG == e)).astype(jnp.float32)
        masks.append(m)
    return masks


def _attn_kernel(q_ref, k_ref, v_ref, ebs_ref, out_ref):
    r = pl.program_id(1)
    scale = DIM_HEAD ** (-0.5)

    qf = q_ref[...] * scale
    base = r * S
    kf = k_ref[pl.ds(base, S), :]
    vf = v_ref[pl.ds(base, S), :]

    masks = _edge_masks()

    stacked = None
    k_shifted = []
    for s in SPATIAL_SHIFTS:
        k_shifted.append(_shift_rows(kf, s))
    for dt in TEMPORAL_SHIFTS:
        t2 = jax.lax.rem(r + dt + T, T)
        k_shifted.append(k_ref[pl.ds(t2 * S, S), :])
    for e in range(TOTAL_EDGES):
        sc = jnp.dot(qf * k_shifted[e], masks[e],
                     preferred_element_type=jnp.float32)
        stacked = sc if stacked is None else stacked + sc

    stacked = stacked + ebs_ref[0]

    m = stacked
    for sh in (HG, 2 * HG, 4 * HG, 8 * HG):
        m = jnp.maximum(m, pltpu.roll(m, sh, 1))
    p = jnp.exp(stacked - m)
    z = p
    for sh in (HG, 2 * HG, 4 * HG, 8 * HG):
        z = z + pltpu.roll(z, sh, 1)
    pn = p * (1.0 / z)

    acc = jnp.zeros((S, HGD), jnp.float32)
    for e, s in enumerate(SPATIAL_SHIFTS):
        vr = _shift_rows(vf, s)
        w = jnp.dot(pn, masks[e].T, preferred_element_type=jnp.float32)
        acc = acc + w * vr
    for i, dt in enumerate(TEMPORAL_SHIFTS):
        e = NUM_SPATIAL + i
        t2 = jax.lax.rem(r + dt + T, T)
        vr = v_ref[pl.ds(t2 * S, S), :]
        w = jnp.dot(pn, masks[e].T, preferred_element_type=jnp.float32)
        acc = acc + w * vr

    out_ref[...] = acc


def _out_kernel(c_ref, wo_ref, bo_ref, o_ref):
    o_ref[...] = jnp.dot(c_ref[...], wo_ref[...],
                         preferred_element_type=jnp.float32) + bo_ref[...]


@functools.partial(jax.jit, static_argnames=())
def kernel(x, Wq, bq, Wk, bk, Wv, bv, Wo, bo, qn_w, kn_w, edge_bias):
    x2 = x.reshape(L, QUERY_DIM)
    bq2 = bq.reshape(1, INNER)
    bk2 = bk.reshape(1, INNER)
    bv2 = bv.reshape(1, INNER)
    bo2 = bo.reshape(1, QUERY_DIM)
    qnw2 = qn_w.reshape(1, INNER)
    knw2 = kn_w.reshape(1, INNER)

    ebs = edge_bias.reshape(G, HG, TOTAL_EDGES).transpose(0, 2, 1).reshape(G, 1, EC)

    n_row_blocks = L // ROW_BLK
    q, k, v = pl.pallas_call(
        _proj_kernel,
        grid=(n_row_blocks,),
        in_specs=[
            pl.BlockSpec((ROW_BLK, QUERY_DIM), lambda i: (i, 0)),
            pl.BlockSpec((QUERY_DIM, INNER), lambda i: (0, 0)),
            pl.BlockSpec((QUERY_DIM, INNER), lambda i: (0, 0)),
            pl.BlockSpec((QUERY_DIM, INNER), lambda i: (0, 0)),
            pl.BlockSpec((1, INNER), lambda i: (0, 0)),
            pl.BlockSpec((1, INNER), lambda i: (0, 0)),
            pl.BlockSpec((1, INNER), lambda i: (0, 0)),
            pl.BlockSpec((1, INNER), lambda i: (0, 0)),
            pl.BlockSpec((1, INNER), lambda i: (0, 0)),
        ],
        out_specs=[
            pl.BlockSpec((ROW_BLK, INNER), lambda i: (i, 0)),
            pl.BlockSpec((ROW_BLK, INNER), lambda i: (i, 0)),
            pl.BlockSpec((ROW_BLK, INNER), lambda i: (i, 0)),
        ],
        out_shape=[jax.ShapeDtypeStruct((L, INNER), jnp.float32)] * 3,
    )(x2, Wq, Wk, Wv, bq2, bk2, bv2, qnw2, knw2)

    combined = pl.pallas_call(
        _attn_kernel,
        grid=(G, T),
        in_specs=[
            pl.BlockSpec((S, HGD), lambda g, r: (r, g)),
            pl.BlockSpec((L, HGD), lambda g, r: (0, g)),
            pl.BlockSpec((L, HGD), lambda g, r: (0, g)),
            pl.BlockSpec((1, 1, EC), lambda g, r: (g, 0, 0)),
        ],
        out_specs=pl.BlockSpec((S, HGD), lambda g, r: (r, g)),
        out_shape=jax.ShapeDtypeStruct((L, INNER), jnp.float32),
    )(q, k, v, ebs)

    out = pl.pallas_call(
        _out_kernel,
        grid=(n_row_blocks,),
        in_specs=[
            pl.BlockSpec((ROW_BLK, INNER), lambda i: (i, 0)),
            pl.BlockSpec((INNER, QUERY_DIM), lambda i: (0, 0)),
            pl.BlockSpec((1, QUERY_DIM), lambda i: (0, 0)),
        ],
        out_specs=pl.BlockSpec((ROW_BLK, QUERY_DIM), lambda i: (i, 0)),
        out_shape=jax.ShapeDtypeStruct((L, QUERY_DIM), jnp.float32),
    )(combined, Wo, bo2)

    return out.reshape(B, L, QUERY_DIM)


# speedup vs baseline: 1.3573x; 1.3573x over previous
import functools
import math

import jax
import jax.numpy as jnp
from jax.experimental import pallas as pl
from jax.experimental.pallas import tpu as pltpu

B = 1
L = 4096
QUERY_DIM = 1024
HEADS = 16
DIM_HEAD = 64
NUM_FRAMES = 8
NUM_SPATIAL = 12
NUM_TEMPORAL = 4
MAX_SPATIAL_LEN = 2048
EPS = 1e-6

T = NUM_FRAMES
S = L // T
INNER = HEADS * DIM_HEAD

HG = 8
G = HEADS // HG
HGD = HG * DIM_HEAD

ROW_BLK = 1024


def _seq_shifts(n, max_len):
    shifts = [0]
    s = 1
    while len(shifts) < n and s < max_len:
        shifts.append(s)
        if len(shifts) < n:
            shifts.append(-s)
        s *= 2
    return shifts[:n]


def _temporal_shifts(n):
    shifts = []
    s = 1
    while len(shifts) < n:
        shifts.append(s)
        if len(shifts) < n:
            shifts.append(-s)
        s *= 2
    return shifts[:n]


SPATIAL_SHIFTS = _seq_shifts(NUM_SPATIAL, MAX_SPATIAL_LEN)
TEMPORAL_SHIFTS = _temporal_shifts(NUM_TEMPORAL)
TOTAL_EDGES = NUM_SPATIAL + NUM_TEMPORAL


def _proj_kernel(x_ref, wq_ref, wk_ref, wv_ref, bq_ref, bk_ref, bv_ref,
                 qnw_ref, knw_ref, q_ref, k_ref, v_ref):
    xb = x_ref[...]
    q = jnp.dot(xb, wq_ref[...], preferred_element_type=jnp.float32) + bq_ref[...]
    k = jnp.dot(xb, wk_ref[...], preferred_element_type=jnp.float32) + bk_ref[...]
    v = jnp.dot(xb, wv_ref[...], preferred_element_type=jnp.float32) + bv_ref[...]
    qm = jnp.mean(q * q, axis=-1, keepdims=True)
    km = jnp.mean(k * k, axis=-1, keepdims=True)
    q_ref[...] = q * jax.lax.rsqrt(qm + EPS) * qnw_ref[...]
    k_ref[...] = k * jax.lax.rsqrt(km + EPS) * knw_ref[...]
    v_ref[...] = v


def _shift_rows(x, s):
    n = x.shape[0]
    s = s % n
    if s == 0:
        return x
    hi = jax.lax.slice_in_dim(x, s, n, axis=0)
    lo = jax.lax.slice_in_dim(x, 0, s, axis=0)
    return jax.lax.concatenate([hi, lo], dimension=0)


def _attn_kernel(q_ref, k_ref, v_ref, eb_ref, out_ref):
    g = pl.program_id(0)
    r = pl.program_id(1)
    scale = DIM_HEAD ** (-0.5)

    qf = q_ref[...] * scale
    base = r * S
    kf = k_ref[pl.ds(base, S), :]
    vf = v_ref[pl.ds(base, S), :]

    d_idx = jax.lax.broadcasted_iota(jnp.int32, (HGD, HG), 0) // DIM_HEAD
    h_idx = jax.lax.broadcasted_iota(jnp.int32, (HGD, HG), 1)
    mask = (d_idx == h_idx).astype(jnp.float32)

    ebg = eb_ref[pl.ds(g * HG, HG), :]

    scores = []
    for s in SPATIAL_SHIFTS:
        kr = _shift_rows(kf, s)
        sc = jnp.dot(qf * kr, mask, preferred_element_type=jnp.float32)
        scores.append(sc)
    for dt in TEMPORAL_SHIFTS:
        t2 = jax.lax.rem(r + dt + T, T)
        kr = k_ref[pl.ds(t2 * S, S), :]
        sc = jnp.dot(qf * kr, mask, preferred_element_type=jnp.float32)
        scores.append(sc)
    for e in range(TOTAL_EDGES):
        scores[e] = scores[e] + ebg[:, e].reshape(1, HG)

    m = scores[0]
    for e in range(1, TOTAL_EDGES):
        m = jnp.maximum(m, scores[e])
    probs = [jnp.exp(sc - m) for sc in scores]
    z = probs[0]
    for e in range(1, TOTAL_EDGES):
        z = z + probs[e]
    inv_z = 1.0 / z

    acc = jnp.zeros((S, HGD), jnp.float32)
    for e, s in enumerate(SPATIAL_SHIFTS):
        vr = _shift_rows(vf, s)
        w = jnp.dot(probs[e], mask.T, preferred_element_type=jnp.float32)
        acc = acc + w * vr
    for i, dt in enumerate(TEMPORAL_SHIFTS):
        e = NUM_SPATIAL + i
        t2 = jax.lax.rem(r + dt + T, T)
        vr = v_ref[pl.ds(t2 * S, S), :]
        w = jnp.dot(probs[e], mask.T, preferred_element_type=jnp.float32)
        acc = acc + w * vr

    nrm = jnp.dot(inv_z, mask.T, preferred_element_type=jnp.float32)
    out_ref[...] = acc * nrm


def _out_kernel(c_ref, wo_ref, bo_ref, o_ref):
    o_ref[...] = jnp.dot(c_ref[...], wo_ref[...],
                         preferred_element_type=jnp.float32) + bo_ref[...]


@functools.partial(jax.jit, static_argnames=())
def kernel(x, Wq, bq, Wk, bk, Wv, bv, Wo, bo, qn_w, kn_w, edge_bias):
    x2 = x.reshape(L, QUERY_DIM)
    bq2 = bq.reshape(1, INNER)
    bk2 = bk.reshape(1, INNER)
    bv2 = bv.reshape(1, INNER)
    bo2 = bo.reshape(1, QUERY_DIM)
    qnw2 = qn_w.reshape(1, INNER)
    knw2 = kn_w.reshape(1, INNER)

    n_row_blocks = L // ROW_BLK
    q, k, v = pl.pallas_call(
        _proj_kernel,
        grid=(n_row_blocks,),
        in_specs=[
            pl.BlockSpec((ROW_BLK, QUERY_DIM), lambda i: (i, 0)),
            pl.BlockSpec((QUERY_DIM, INNER), lambda i: (0, 0)),
            pl.BlockSpec((QUERY_DIM, INNER), lambda i: (0, 0)),
            pl.BlockSpec((QUERY_DIM, INNER), lambda i: (0, 0)),
            pl.BlockSpec((1, INNER), lambda i: (0, 0)),
            pl.BlockSpec((1, INNER), lambda i: (0, 0)),
            pl.BlockSpec((1, INNER), lambda i: (0, 0)),
            pl.BlockSpec((1, INNER), lambda i: (0, 0)),
            pl.BlockSpec((1, INNER), lambda i: (0, 0)),
        ],
        out_specs=[
            pl.BlockSpec((ROW_BLK, INNER), lambda i: (i, 0)),
            pl.BlockSpec((ROW_BLK, INNER), lambda i: (i, 0)),
            pl.BlockSpec((ROW_BLK, INNER), lambda i: (i, 0)),
        ],
        out_shape=[jax.ShapeDtypeStruct((L, INNER), jnp.float32)] * 3,
    )(x2, Wq, Wk, Wv, bq2, bk2, bv2, qnw2, knw2)

    combined = pl.pallas_call(
        _attn_kernel,
        grid=(G, T),
        in_specs=[
            pl.BlockSpec((S, HGD), lambda g, r: (r, g)),
            pl.BlockSpec((L, HGD), lambda g, r: (0, g)),
            pl.BlockSpec((L, HGD), lambda g, r: (0, g)),
            pl.BlockSpec((HEADS, TOTAL_EDGES), lambda g, r: (0, 0)),
        ],
        out_specs=pl.BlockSpec((S, HGD), lambda g, r: (r, g)),
        out_shape=jax.ShapeDtypeStruct((L, INNER), jnp.float32),
    )(q, k, v, edge_bias)

    out = pl.pallas_call(
        _out_kernel,
        grid=(n_row_blocks,),
        in_specs=[
            pl.BlockSpec((ROW_BLK, INNER), lambda i: (i, 0)),
            pl.BlockSpec((INNER, QUERY_DIM), lambda i: (0, 0)),
            pl.BlockSpec((1, QUERY_DIM), lambda i: (0, 0)),
        ],
        out_specs=pl.BlockSpec((ROW_BLK, QUERY_DIM), lambda i: (i, 0)),
        out_shape=jax.ShapeDtypeStruct((L, QUERY_DIM), jnp.float32),
    )(combined, Wo, bo2)

    return out.reshape(B, L, QUERY_DIM)


# speedup vs baseline: 1.4438x; 1.0637x over previous
import functools
import math

import jax
import jax.numpy as jnp
from jax.experimental import pallas as pl
from jax.experimental.pallas import tpu as pltpu

B = 1
L = 4096
QUERY_DIM = 1024
HEADS = 16
DIM_HEAD = 64
NUM_FRAMES = 8
NUM_SPATIAL = 12
NUM_TEMPORAL = 4
MAX_SPATIAL_LEN = 2048
EPS = 1e-6

T = NUM_FRAMES
S = L // T
INNER = HEADS * DIM_HEAD

HG = 16
G = HEADS // HG
HGD = HG * DIM_HEAD

ROW_BLK = 1024


def _seq_shifts(n, max_len):
    shifts = [0]
    s = 1
    while len(shifts) < n and s < max_len:
        shifts.append(s)
        if len(shifts) < n:
            shifts.append(-s)
        s *= 2
    return shifts[:n]


def _temporal_shifts(n):
    shifts = []
    s = 1
    while len(shifts) < n:
        shifts.append(s)
        if len(shifts) < n:
            shifts.append(-s)
        s *= 2
    return shifts[:n]


SPATIAL_SHIFTS = _seq_shifts(NUM_SPATIAL, MAX_SPATIAL_LEN)
TEMPORAL_SHIFTS = _temporal_shifts(NUM_TEMPORAL)
TOTAL_EDGES = NUM_SPATIAL + NUM_TEMPORAL


def _proj_kernel(x_ref, wq_ref, wk_ref, wv_ref, bq_ref, bk_ref, bv_ref,
                 qnw_ref, knw_ref, q_ref, k_ref, v_ref):
    xb = x_ref[...]
    q = jnp.dot(xb, wq_ref[...], preferred_element_type=jnp.float32) + bq_ref[...]
    k = jnp.dot(xb, wk_ref[...], preferred_element_type=jnp.float32) + bk_ref[...]
    v = jnp.dot(xb, wv_ref[...], preferred_element_type=jnp.float32) + bv_ref[...]
    qm = jnp.mean(q * q, axis=-1, keepdims=True)
    km = jnp.mean(k * k, axis=-1, keepdims=True)
    q_ref[...] = q * jax.lax.rsqrt(qm + EPS) * qnw_ref[...]
    k_ref[...] = k * jax.lax.rsqrt(km + EPS) * knw_ref[...]
    v_ref[...] = v


def _shift_rows(x, s):
    n = x.shape[0]
    s = s % n
    if s == 0:
        return x
    hi = jax.lax.slice_in_dim(x, s, n, axis=0)
    lo = jax.lax.slice_in_dim(x, 0, s, axis=0)
    return jax.lax.concatenate([hi, lo], dimension=0)


def _attn_kernel(q_ref, k_ref, v_ref, eb_ref, wo_ref, bo_ref, out_ref):
    r = pl.program_id(0)
    scale = DIM_HEAD ** (-0.5)

    qf = q_ref[...] * scale
    base = r * S
    kf = k_ref[pl.ds(base, S), :]
    vf = v_ref[pl.ds(base, S), :]

    d_idx = jax.lax.broadcasted_iota(jnp.int32, (HGD, HG), 0) // DIM_HEAD
    h_idx = jax.lax.broadcasted_iota(jnp.int32, (HGD, HG), 1)
    mask = (d_idx == h_idx).astype(jnp.float32)

    ebg = eb_ref[...]

    scores = []
    for s in SPATIAL_SHIFTS:
        kr = _shift_rows(kf, s)
        sc = jnp.dot(qf * kr, mask, preferred_element_type=jnp.float32)
        scores.append(sc)
    for dt in TEMPORAL_SHIFTS:
        t2 = jax.lax.rem(r + dt + T, T)
        kr = k_ref[pl.ds(t2 * S, S), :]
        sc = jnp.dot(qf * kr, mask, preferred_element_type=jnp.float32)
        scores.append(sc)
    for e in range(TOTAL_EDGES):
        scores[e] = scores[e] + ebg[:, e].reshape(1, HG)

    m = scores[0]
    for e in range(1, TOTAL_EDGES):
        m = jnp.maximum(m, scores[e])
    probs = [jnp.exp(sc - m) for sc in scores]
    z = probs[0]
    for e in range(1, TOTAL_EDGES):
        z = z + probs[e]
    inv_z = 1.0 / z

    acc = jnp.zeros((S, HGD), jnp.float32)
    for e, s in enumerate(SPATIAL_SHIFTS):
        vr = _shift_rows(vf, s)
        w = jnp.dot(probs[e], mask.T, preferred_element_type=jnp.float32)
        acc = acc + w * vr
    for i, dt in enumerate(TEMPORAL_SHIFTS):
        e = NUM_SPATIAL + i
        t2 = jax.lax.rem(r + dt + T, T)
        vr = v_ref[pl.ds(t2 * S, S), :]
        w = jnp.dot(probs[e], mask.T, preferred_element_type=jnp.float32)
        acc = acc + w * vr

    nrm = jnp.dot(inv_z, mask.T, preferred_element_type=jnp.float32)
    out_ref[...] = jnp.dot(acc * nrm, wo_ref[...],
                           preferred_element_type=jnp.float32) + bo_ref[...]


@functools.partial(jax.jit, static_argnames=())
def kernel(x, Wq, bq, Wk, bk, Wv, bv, Wo, bo, qn_w, kn_w, edge_bias):
    x2 = x.reshape(L, QUERY_DIM)
    bq2 = bq.reshape(1, INNER)
    bk2 = bk.reshape(1, INNER)
    bv2 = bv.reshape(1, INNER)
    bo2 = bo.reshape(1, QUERY_DIM)
    qnw2 = qn_w.reshape(1, INNER)
    knw2 = kn_w.reshape(1, INNER)

    n_row_blocks = L // ROW_BLK
    q, k, v = pl.pallas_call(
        _proj_kernel,
        grid=(n_row_blocks,),
        in_specs=[
            pl.BlockSpec((ROW_BLK, QUERY_DIM), lambda i: (i, 0)),
            pl.BlockSpec((QUERY_DIM, INNER), lambda i: (0, 0)),
            pl.BlockSpec((QUERY_DIM, INNER), lambda i: (0, 0)),
            pl.BlockSpec((QUERY_DIM, INNER), lambda i: (0, 0)),
            pl.BlockSpec((1, INNER), lambda i: (0, 0)),
            pl.BlockSpec((1, INNER), lambda i: (0, 0)),
            pl.BlockSpec((1, INNER), lambda i: (0, 0)),
            pl.BlockSpec((1, INNER), lambda i: (0, 0)),
            pl.BlockSpec((1, INNER), lambda i: (0, 0)),
        ],
        out_specs=[
            pl.BlockSpec((ROW_BLK, INNER), lambda i: (i, 0)),
            pl.BlockSpec((ROW_BLK, INNER), lambda i: (i, 0)),
            pl.BlockSpec((ROW_BLK, INNER), lambda i: (i, 0)),
        ],
        out_shape=[jax.ShapeDtypeStruct((L, INNER), jnp.float32)] * 3,
    )(x2, Wq, Wk, Wv, bq2, bk2, bv2, qnw2, knw2)

    out = pl.pallas_call(
        _attn_kernel,
        grid=(T,),
        in_specs=[
            pl.BlockSpec((S, INNER), lambda r: (r, 0)),
            pl.BlockSpec((L, INNER), lambda r: (0, 0)),
            pl.BlockSpec((L, INNER), lambda r: (0, 0)),
            pl.BlockSpec((HEADS, TOTAL_EDGES), lambda r: (0, 0)),
            pl.BlockSpec((INNER, QUERY_DIM), lambda r: (0, 0)),
            pl.BlockSpec((1, QUERY_DIM), lambda r: (0, 0)),
        ],
        out_specs=pl.BlockSpec((S, QUERY_DIM), lambda r: (r, 0)),
        out_shape=jax.ShapeDtypeStruct((L, QUERY_DIM), jnp.float32),
    )(q, k, v, edge_bias, Wo, bo2)

    return out.reshape(B, L, QUERY_DIM)


# speedup vs baseline: 1.6091x; 1.1144x over previous
import functools
import math

import jax
import jax.numpy as jnp
from jax.experimental import pallas as pl
from jax.experimental.pallas import tpu as pltpu

B = 1
L = 4096
QUERY_DIM = 1024
HEADS = 16
DIM_HEAD = 64
NUM_FRAMES = 8
NUM_SPATIAL = 12
NUM_TEMPORAL = 4
MAX_SPATIAL_LEN = 2048
EPS = 1e-6

T = NUM_FRAMES
S = L // T
INNER = HEADS * DIM_HEAD


def _seq_shifts(n, max_len):
    shifts = [0]
    s = 1
    while len(shifts) < n and s < max_len:
        shifts.append(s)
        if len(shifts) < n:
            shifts.append(-s)
        s *= 2
    return shifts[:n]


def _temporal_shifts(n):
    shifts = []
    s = 1
    while len(shifts) < n:
        shifts.append(s)
        if len(shifts) < n:
            shifts.append(-s)
        s *= 2
    return shifts[:n]


SPATIAL_SHIFTS = _seq_shifts(NUM_SPATIAL, MAX_SPATIAL_LEN)
TEMPORAL_SHIFTS = _temporal_shifts(NUM_TEMPORAL)
TOTAL_EDGES = NUM_SPATIAL + NUM_TEMPORAL


def _shift_rows(x, s):
    n = x.shape[0]
    s = s % n
    if s == 0:
        return x
    hi = jax.lax.slice_in_dim(x, s, n, axis=0)
    lo = jax.lax.slice_in_dim(x, 0, s, axis=0)
    return jax.lax.concatenate([hi, lo], dimension=0)


def _fused_kernel(x_ref, wq_ref, wk_ref, wv_ref, bq_ref, bk_ref, bv_ref,
                  qnw_ref, knw_ref, eb_ref, wo_ref, bo_ref, out_ref,
                  q_scr, k_scr, v_scr):
    i = pl.program_id(0)

    @pl.when(i < T)
    def _project():
        xb = x_ref[...]
        q = jnp.dot(xb, wq_ref[...], preferred_element_type=jnp.float32) + bq_ref[...]
        k = jnp.dot(xb, wk_ref[...], preferred_element_type=jnp.float32) + bk_ref[...]
        v = jnp.dot(xb, wv_ref[...], preferred_element_type=jnp.float32) + bv_ref[...]
        qm = jnp.mean(q * q, axis=-1, keepdims=True)
        km = jnp.mean(k * k, axis=-1, keepdims=True)
        q = q * jax.lax.rsqrt(qm + EPS) * qnw_ref[...]
        k = k * jax.lax.rsqrt(km + EPS) * knw_ref[...]
        base = i * S
        q_scr[pl.ds(base, S), :] = q.astype(jnp.bfloat16)
        k_scr[pl.ds(base, S), :] = k.astype(jnp.bfloat16)
        v_scr[pl.ds(base, S), :] = v.astype(jnp.bfloat16)

    @pl.when(i >= T)
    def _attend():
        r = i - T
        scale = DIM_HEAD ** (-0.5)
        base = r * S
        qf = q_scr[pl.ds(base, S), :].astype(jnp.float32) * scale
        kf = k_scr[pl.ds(base, S), :].astype(jnp.float32)
        vf = v_scr[pl.ds(base, S), :].astype(jnp.float32)

        d_idx = jax.lax.broadcasted_iota(jnp.int32, (INNER, HEADS), 0) // DIM_HEAD
        h_idx = jax.lax.broadcasted_iota(jnp.int32, (INNER, HEADS), 1)
        mask = (d_idx == h_idx).astype(jnp.float32)

        ebg = eb_ref[...]

        scores = []
        for s in SPATIAL_SHIFTS:
            kr = _shift_rows(kf, s)
            sc = jnp.dot(qf * kr, mask, preferred_element_type=jnp.float32)
            scores.append(sc)
        for dt in TEMPORAL_SHIFTS:
            t2 = jax.lax.rem(r + dt + T, T)
            kr = k_scr[pl.ds(t2 * S, S), :].astype(jnp.float32)
            sc = jnp.dot(qf * kr, mask, preferred_element_type=jnp.float32)
            scores.append(sc)
        for e in range(TOTAL_EDGES):
            scores[e] = scores[e] + ebg[:, e].reshape(1, HEADS)

        m = scores[0]
        for e in range(1, TOTAL_EDGES):
            m = jnp.maximum(m, scores[e])
        probs = [jnp.exp(sc - m) for sc in scores]
        z = probs[0]
        for e in range(1, TOTAL_EDGES):
            z = z + probs[e]
        inv_z = 1.0 / z

        acc = jnp.zeros((S, INNER), jnp.float32)
        for e, s in enumerate(SPATIAL_SHIFTS):
            vr = _shift_rows(vf, s)
            w = jnp.dot(probs[e], mask.T, preferred_element_type=jnp.float32)
            acc = acc + w * vr
        for j, dt in enumerate(TEMPORAL_SHIFTS):
            e = NUM_SPATIAL + j
            t2 = jax.lax.rem(r + dt + T, T)
            vr = v_scr[pl.ds(t2 * S, S), :].astype(jnp.float32)
            w = jnp.dot(probs[e], mask.T, preferred_element_type=jnp.float32)
            acc = acc + w * vr

        nrm = jnp.dot(inv_z, mask.T, preferred_element_type=jnp.float32)
        out_ref[...] = jnp.dot(acc * nrm, wo_ref[...],
                               preferred_element_type=jnp.float32) + bo_ref[...]


@functools.partial(jax.jit, static_argnames=())
def kernel(x, Wq, bq, Wk, bk, Wv, bv, Wo, bo, qn_w, kn_w, edge_bias):
    x2 = x.reshape(L, QUERY_DIM)
    bq2 = bq.reshape(1, INNER)
    bk2 = bk.reshape(1, INNER)
    bv2 = bv.reshape(1, INNER)
    bo2 = bo.reshape(1, QUERY_DIM)
    qnw2 = qn_w.reshape(1, INNER)
    knw2 = kn_w.reshape(1, INNER)

    out = pl.pallas_call(
        _fused_kernel,
        grid=(2 * T,),
        in_specs=[
            pl.BlockSpec((S, QUERY_DIM), lambda i: (jnp.minimum(i, T - 1), 0)),
            pl.BlockSpec((QUERY_DIM, INNER), lambda i: (0, 0)),
            pl.BlockSpec((QUERY_DIM, INNER), lambda i: (0, 0)),
            pl.BlockSpec((QUERY_DIM, INNER), lambda i: (0, 0)),
            pl.BlockSpec((1, INNER), lambda i: (0, 0)),
            pl.BlockSpec((1, INNER), lambda i: (0, 0)),
            pl.BlockSpec((1, INNER), lambda i: (0, 0)),
            pl.BlockSpec((1, INNER), lambda i: (0, 0)),
            pl.BlockSpec((1, INNER), lambda i: (0, 0)),
            pl.BlockSpec((HEADS, TOTAL_EDGES), lambda i: (0, 0)),
            pl.BlockSpec((INNER, QUERY_DIM), lambda i: (0, 0)),
            pl.BlockSpec((1, QUERY_DIM), lambda i: (0, 0)),
        ],
        out_specs=pl.BlockSpec((S, QUERY_DIM),
                               lambda i: (jnp.maximum(i - T, 0), 0)),
        out_shape=jax.ShapeDtypeStruct((L, QUERY_DIM), jnp.float32),
        scratch_shapes=[
            pltpu.VMEM((L, INNER), jnp.bfloat16),
            pltpu.VMEM((L, INNER), jnp.bfloat16),
            pltpu.VMEM((L, INNER), jnp.bfloat16),
        ],
    )(x2, Wq, Wk, Wv, bq2, bk2, bv2, qnw2, knw2, edge_bias, Wo, bo2)

    return out.reshape(B, L, QUERY_DIM)
